# baseline (speedup 1.0000x reference)
import functools

import jax
import jax.numpy as jnp
from jax import lax
from jax.experimental import pallas as pl
from jax.experimental.pallas import tpu as pltpu
from jax.experimental.pallas import tpu_sc as plsc

D = 32
NSAMP = 5
NC, NSUB, L = 2, 16, 16
NW = NC * NSUB
CHUNK = 128


@functools.cache
def _build(B):
    BW = B // NW
    CW = BW * NSAMP
    NCH_T = BW // CHUNK
    NCH_C = CW // CHUNK
    G = BW // L

    mesh = plsc.VectorSubcoreMesh(core_axis_name="c", subcore_axis_name="s")

    @functools.partial(
        pl.kernel,
        out_type=jax.ShapeDtypeStruct((B, NSAMP), jnp.float32),
        mesh=mesh,
        compiler_params=pltpu.CompilerParams(
            use_tc_tiling_on_sc=False, needs_layout_passes=False),
        scratch_types=[
            pltpu.VMEM((NCH_T, CHUNK), jnp.int32),
            pltpu.VMEM((NCH_C, CHUNK), jnp.int32),
            pltpu.VMEM((BW, D), jnp.float32),
            pltpu.VMEM((CW, D), jnp.float32),
            pltpu.VMEM((BW, NSAMP), jnp.float32),
            pltpu.SemaphoreType.DMA,
        ],
    )
    def k(tgt_hbm, ctx_hbm, ttab_hbm, ctab_hbm, out_hbm,
          tidx_v, cidx_v, trows_v, crows_v, out_v, sem):
        wid = lax.axis_index("s") * NC + lax.axis_index("c")
        tb = wid * BW
        cb = wid * CW

        for j in range(NCH_T):
            pltpu.sync_copy(tgt_hbm.at[pl.ds(tb + j * CHUNK, CHUNK)],
                            tidx_v.at[j])
        for j in range(NCH_C):
            pltpu.sync_copy(ctx_hbm.at[pl.ds(cb + j * CHUNK, CHUNK)],
                            cidx_v.at[j])

        copies = []
        for j in range(NCH_T):
            copies.append(pltpu.async_copy(
                ttab_hbm.at[tidx_v.at[j]],
                trows_v.at[pl.ds(j * CHUNK, CHUNK), :], sem))
        for j in range(NCH_C):
            copies.append(pltpu.async_copy(
                ctab_hbm.at[cidx_v.at[j]],
                crows_v.at[pl.ds(j * CHUNK, CHUNK), :], sem))
        for c in copies:
            c.wait()

        iota = lax.iota(jnp.int32, 16)

        def group(g, carry):
            wrow = g * L + iota
            crow = g * (L * NSAMP) + iota * NSAMP
            accs = [jnp.zeros((L,), jnp.float32) for _ in range(NSAMP)]
            for d in range(D):
                dcol = jnp.full((L,), d, jnp.int32)
                wv = plsc.load_gather(trows_v, [wrow, dcol])
                for c in range(NSAMP):
                    cv = plsc.load_gather(crows_v, [crow + c, dcol])
                    accs[c] = accs[c] + wv * cv
            for c in range(NSAMP):
                plsc.store_scatter(
                    out_v, [wrow, jnp.full((L,), c, jnp.int32)], accs[c])
            return carry

        lax.fori_loop(0, G, group, 0)
        pltpu.sync_copy(out_v, out_hbm.at[pl.ds(tb, BW), :])

    return k


def kernel(target, context, target_table, context_table):
    B = target.shape[0]
    k = _build(B)
    return k(target.reshape(-1), context.reshape(-1),
             target_table, context_table)
